# baseline (speedup 1.0000x reference)
import functools

import jax
import jax.numpy as jnp
from jax import lax
from jax.experimental import pallas as pl
from jax.experimental.pallas import tpu as pltpu
from jax.experimental.pallas import tpu_sc as plsc

T = 2048
C = 1024
E = 8
H = 4096
BLK = 128
NBLK = 40
NPAD = NBLK * BLK
NPF = NPAD - 2 * T
NH = 4
HT = H // NH
NC, NS = 2, 16
NW = NC * NS
PER = NPAD // NW
WCH = 32
NCH = PER // WCH
WLN = 128


def _gating_body(x_ref, gw_ref, gb_ref, dest_ref, w_ref, blk_ref):
    logits = lax.dot_general(
        x_ref[...].astype(jnp.bfloat16), gw_ref[...].astype(jnp.bfloat16),
        (((1,), (0,)), ((), ())),
        preferred_element_type=jnp.float32) + gb_ref[...]

    it8 = lax.broadcasted_iota(jnp.int32, (T, E), 1)
    m1 = jnp.max(logits, axis=1, keepdims=True)
    i1 = jnp.min(jnp.where(logits == m1, it8, E), axis=1, keepdims=True)
    l2 = jnp.where(it8 == i1, -jnp.inf, logits)
    m2 = jnp.max(l2, axis=1, keepdims=True)
    i2 = jnp.min(jnp.where(l2 == m2, it8, E), axis=1, keepdims=True)

    t = jnp.exp(m2 - m1)
    nw0 = 1.0 / (1.0 + t)
    nw1 = t / (1.0 + t)

    oh0 = (it8 == i1).astype(jnp.float32)
    oh1 = (it8 == i2).astype(jnp.float32)

    r_io = lax.broadcasted_iota(jnp.int32, (T, T), 0)
    c_io = lax.broadcasted_iota(jnp.int32, (T, T), 1)
    tri = (r_io > c_io).astype(jnp.float32)
    dots = (((1,), (0,)), ((), ()))
    pos0 = lax.dot_general(tri, oh0, dots, preferred_element_type=jnp.float32)
    pos1 = lax.dot_general(tri, oh1, dots, preferred_element_type=jnp.float32)

    c0 = jnp.sum(oh0, axis=0, keepdims=True)
    c1 = jnp.sum(oh1, axis=0, keepdims=True)
    cnt = c0 + c1
    pad = jnp.ceil(cnt / BLK) * BLK

    e_r = lax.broadcasted_iota(jnp.int32, (E, E), 0)
    e_c = lax.broadcasted_iota(jnp.int32, (E, E), 1)
    upper = (e_r < e_c).astype(jnp.float32)
    off = lax.dot_general(pad, upper, dots, preferred_element_type=jnp.float32)

    dest0 = jnp.sum(oh0 * (off + pos0), axis=1, keepdims=True)
    dest1 = jnp.sum(oh1 * (off + c0 + pos1), axis=1, keepdims=True)

    q = pad - cnt
    qx = lax.dot_general(q, upper, dots, preferred_element_type=jnp.float32)
    q8 = jnp.sum(q, axis=1, keepdims=True)
    j = lax.broadcasted_iota(jnp.int32, (T, 1), 0).astype(jnp.float32)
    in_e = ((j >= qx) & (j < qx + q)).astype(jnp.float32)
    dp_in = jnp.sum(in_e * (off + cnt + j - qx), axis=1, keepdims=True)
    dest_pad = jnp.where(j < q8, dp_in, float(2 * T) + j)

    dest_ref[...] = jnp.concatenate(
        [dest0, dest1, dest_pad], axis=1).astype(jnp.int32)
    w_ref[...] = jnp.concatenate(
        [nw0, nw1, jnp.zeros_like(nw0)], axis=1)

    sb = lax.broadcasted_iota(jnp.int32, (NBLK, 1), 0).astype(jnp.float32) * BLK
    blk = jnp.sum((sb >= off).astype(jnp.int32), axis=1, keepdims=True) - 1
    blk_ref[...] = jnp.clip(blk, 0, E - 1)


def _gating(x_flat, gate_w, gate_b):
    return pl.pallas_call(
        _gating_body,
        out_shape=(
            jax.ShapeDtypeStruct((T, 3), jnp.int32),
            jax.ShapeDtypeStruct((T, 3), jnp.float32),
            jax.ShapeDtypeStruct((NBLK, 1), jnp.int32),
        ),
    )(x_flat, gate_w, gate_b.reshape(1, E))


def _dispatch(xb3, dest3, destf, tok3, w_rep):
    mesh = plsc.VectorSubcoreMesh(core_axis_name="c", subcore_axis_name="s")

    @functools.partial(
        pl.kernel, mesh=mesh,
        out_type=(jax.ShapeDtypeStruct((NPAD, 4, 128), jnp.int32),
                  jax.ShapeDtypeStruct((NPAD, WLN), jnp.float32)),
        scratch_types=[pltpu.VMEM((NCH, WCH), jnp.int32),
                       pltpu.VMEM((NCH, WCH), jnp.int32),
                       pltpu.VMEM((PER,), jnp.int32),
                       pltpu.VMEM((WCH, 4, 128), jnp.int32),
                       pltpu.VMEM((WCH, 4, 128), jnp.int32),
                       pltpu.VMEM((PER, WLN), jnp.float32),
                       pltpu.SemaphoreType.DMA,
                       pltpu.SemaphoreType.DMA,
                       pltpu.SemaphoreType.DMA,
                       pltpu.SemaphoreType.DMA,
                       pltpu.SemaphoreType.DMA])
    def k(x_hbm, dest_hbm, destf_hbm, tok_hbm, w_hbm, xs_hbm, sw_hbm,
          tid_v, did_v, dif_v, buf_a, buf_b, wbuf, ga, gb, sa, sb_, ws):
        wid = lax.axis_index("s") * NC + lax.axis_index("c")
        pltpu.sync_copy(tok_hbm.at[wid], tid_v)
        pltpu.sync_copy(dest_hbm.at[wid], did_v)
        pltpu.sync_copy(destf_hbm.at[wid], dif_v)
        pltpu.sync_copy(w_hbm.at[wid], wbuf)
        wsc = pltpu.async_copy(wbuf, sw_hbm.at[dif_v], ws)

        bufs = (buf_a, buf_b)
        gsem = (ga, gb)
        ssem = (sa, sb_)
        gh = [None] * NCH
        sh = [None] * NCH
        gh[0] = pltpu.async_copy(x_hbm.at[tid_v.at[0]], bufs[0], gsem[0])
        for kk in range(NCH):
            cur = kk % 2
            gh[kk].wait()
            if kk + 1 < NCH:
                nxt = (kk + 1) % 2
                if kk >= 1:
                    sh[kk - 1].wait()
                gh[kk + 1] = pltpu.async_copy(
                    x_hbm.at[tid_v.at[kk + 1]], bufs[nxt], gsem[nxt])
            sh[kk] = pltpu.async_copy(bufs[cur], xs_hbm.at[did_v.at[kk]],
                                      ssem[cur])
        sh[NCH - 2].wait()
        sh[NCH - 1].wait()
        wsc.wait()

    return k(xb3, dest3, destf, tok3, w_rep)


_GA1, _GA2, _GA3 = 0.254829592, -0.284496736, 1.421413741
_GA4, _GA5, _GP = -1.453152027, 1.061405429, 0.3275911


def _erf(z):
    s = jnp.sign(z)
    az = jnp.abs(z)
    t = 1.0 / (1.0 + _GP * az)
    poly = ((((_GA5 * t + _GA4) * t + _GA3) * t + _GA2) * t + _GA1) * t
    return s * (1.0 - poly * jnp.exp(-az * az))


def _ffn_body(blk_ref, xs_ref, sw_ref, w1_ref, b1_ref, w2_ref, b2_ref,
              ys_ref, acc_ref):
    h_idx = pl.program_id(0)
    b_idx = pl.program_id(1)
    hpre = lax.dot_general(
        xs_ref[...], w1_ref[0], (((1,), (0,)), ((), ())),
        preferred_element_type=jnp.float32) + b1_ref[0, 0]
    act = 0.5 * hpre * (1.0 + _erf(hpre * 0.7071067811865476))
    part = lax.dot_general(
        act.astype(jnp.bfloat16), w2_ref[0], (((1,), (0,)), ((), ())),
        preferred_element_type=jnp.float32)

    sl = pl.ds(b_idx * BLK, BLK)

    @pl.when(h_idx == 0)
    def _():
        acc_ref[sl, :] = part + b2_ref[0]

    @pl.when(h_idx > 0)
    def _():
        acc_ref[sl, :] += part

    @pl.when(h_idx == NH - 1)
    def _():
        ys_ref[...] = acc_ref[sl, :] * sw_ref[...][:, :1]


def _ffn(blk_e, xs, sw, w1b, b1, w2b, b2):
    grid_spec = pltpu.PrefetchScalarGridSpec(
        num_scalar_prefetch=1,
        grid=(NH, NBLK),
        in_specs=[
            pl.BlockSpec((BLK, C), lambda h, b, be: (b, 0)),
            pl.BlockSpec((BLK, WLN), lambda h, b, be: (b, 0)),
            pl.BlockSpec((1, C, HT), lambda h, b, be: (be[b], 0, h)),
            pl.BlockSpec((1, 1, 1, HT), lambda h, b, be: (be[b], h, 0, 0)),
            pl.BlockSpec((1, HT, C), lambda h, b, be: (be[b], h, 0)),
            pl.BlockSpec((1, 1, C), lambda h, b, be: (be[b], 0, 0)),
        ],
        out_specs=pl.BlockSpec(
            (BLK, C), lambda h, b, be: (jnp.where(h == NH - 1, b, 0), 0)),
        scratch_shapes=[pltpu.VMEM((NPAD, C), jnp.float32)],
    )
    return pl.pallas_call(
        _ffn_body,
        grid_spec=grid_spec,
        out_shape=jax.ShapeDtypeStruct((NPAD, C), jnp.float32),
    )(blk_e, xs, sw, w1b, b1.reshape(E, NH, 1, HT), w2b, b2.reshape(E, 1, C))


def _combine(ys, d0, d1):
    mesh = plsc.VectorSubcoreMesh(core_axis_name="c", subcore_axis_name="s")

    @functools.partial(
        pl.kernel, mesh=mesh,
        out_type=jax.ShapeDtypeStruct((T, C), jnp.float32),
        scratch_types=[pltpu.VMEM((WCH,), jnp.int32),
                       pltpu.VMEM((WCH,), jnp.int32),
                       pltpu.VMEM((WCH, C), jnp.float32),
                       pltpu.VMEM((WCH, C), jnp.float32),
                       pltpu.SemaphoreType.DMA])
    def k(ys_hbm, d0_hbm, d1_hbm, out_hbm, i0_v, i1_v, g0, g1, sem):
        wid = lax.axis_index("s") * NC + lax.axis_index("c")
        per = T // NW
        for kk in range(per // WCH):
            base = wid * per + kk * WCH
            pltpu.sync_copy(d0_hbm.at[0, pl.ds(base, WCH)], i0_v)
            pltpu.sync_copy(d1_hbm.at[0, pl.ds(base, WCH)], i1_v)
            pltpu.async_copy(ys_hbm.at[i0_v], g0, sem).wait()
            pltpu.async_copy(ys_hbm.at[i1_v], g1, sem).wait()

            @pl.loop(0, WCH)
            def _(r):
                for cc in range(0, C, 16):
                    g0[r, pl.ds(cc, 16)] = (g0[r, pl.ds(cc, 16)]
                                            + g1[r, pl.ds(cc, 16)])

            pltpu.sync_copy(g0, out_hbm.at[pl.ds(base, WCH)])

    return k(ys, d0, d1)


def kernel(x, gate_w, gate_b, w1, b1, w2, b2):
    B, Tb, Cb = x.shape
    x_flat = x.reshape(T, C)

    dest_out, w_out, blk_e = _gating(x_flat, gate_w, gate_b)

    dest_flat = jnp.concatenate(
        [dest_out[:, 0], dest_out[:, 1], dest_out[:NPF, 2]])
    tok = jnp.arange(T, dtype=jnp.int32)
    tok_flat = jnp.concatenate([tok, tok, jnp.zeros((NPF,), jnp.int32)])
    w_flat = jnp.concatenate(
        [w_out[:, 0], w_out[:, 1], jnp.zeros((NPF,), jnp.float32)])

    xb3 = lax.bitcast_convert_type(
        x_flat.astype(jnp.bfloat16).reshape(T, 512, 2),
        jnp.int32).reshape(T, 4, 128)
    xs, sw = _dispatch(xb3,
                       dest_flat.reshape(NW, NCH, WCH),
                       dest_flat.reshape(NW, PER),
                       tok_flat.reshape(NW, NCH, WCH),
                       jnp.broadcast_to(w_flat.reshape(NPAD, 1),
                                        (NPAD, WLN)).reshape(NW, PER, WLN))

    xs_bf = lax.bitcast_convert_type(
        xs.reshape(NPAD, 512), jnp.bfloat16).reshape(NPAD, C)
    ys = _ffn(blk_e.reshape(NBLK), xs_bf, sw,
              w1.astype(jnp.bfloat16), b1, w2.astype(jnp.bfloat16), b2)

    d0 = dest_out[:, 0].reshape(1, T)
    d1 = dest_out[:, 1].reshape(1, T)
    out = _combine(ys, d0, d1)
    return out.reshape(B, Tb, Cb)


# speedup vs baseline: 1.2722x; 1.2722x over previous
import functools

import jax
import jax.numpy as jnp
from jax import lax
from jax.experimental import pallas as pl
from jax.experimental.pallas import tpu as pltpu
from jax.experimental.pallas import tpu_sc as plsc

T = 2048
C = 1024
E = 8
H = 4096
BLK = 256
NBLK = 24
NPAD = NBLK * BLK
NPF = NPAD - 2 * T
NH = 4
HT = H // NH
NC, NS = 2, 16
NW = NC * NS
PER = NPAD // NW
WCH = 32
NCH = PER // WCH
WLN = 128


def _gating_body(x_ref, gw_ref, gb_ref, dest_ref, w_ref, blk_ref):
    logits = lax.dot_general(
        x_ref[...].astype(jnp.bfloat16), gw_ref[...].astype(jnp.bfloat16),
        (((1,), (0,)), ((), ())),
        preferred_element_type=jnp.float32) + gb_ref[...]

    it8 = lax.broadcasted_iota(jnp.int32, (T, E), 1)
    m1 = jnp.max(logits, axis=1, keepdims=True)
    i1 = jnp.min(jnp.where(logits == m1, it8, E), axis=1, keepdims=True)
    l2 = jnp.where(it8 == i1, -jnp.inf, logits)
    m2 = jnp.max(l2, axis=1, keepdims=True)
    i2 = jnp.min(jnp.where(l2 == m2, it8, E), axis=1, keepdims=True)

    t = jnp.exp(m2 - m1)
    nw0 = 1.0 / (1.0 + t)
    nw1 = t / (1.0 + t)

    oh0 = (it8 == i1).astype(jnp.float32)
    oh1 = (it8 == i2).astype(jnp.float32)

    r_io = lax.broadcasted_iota(jnp.int32, (T, T), 0)
    c_io = lax.broadcasted_iota(jnp.int32, (T, T), 1)
    tri = (r_io > c_io).astype(jnp.float32)
    dots = (((1,), (0,)), ((), ()))
    pos0 = lax.dot_general(tri, oh0, dots, preferred_element_type=jnp.float32)
    pos1 = lax.dot_general(tri, oh1, dots, preferred_element_type=jnp.float32)

    c0 = jnp.sum(oh0, axis=0, keepdims=True)
    c1 = jnp.sum(oh1, axis=0, keepdims=True)
    cnt = c0 + c1
    pad = jnp.ceil(cnt / BLK) * BLK

    e_r = lax.broadcasted_iota(jnp.int32, (E, E), 0)
    e_c = lax.broadcasted_iota(jnp.int32, (E, E), 1)
    upper = (e_r < e_c).astype(jnp.float32)
    off = lax.dot_general(pad, upper, dots, preferred_element_type=jnp.float32)

    dest0 = jnp.sum(oh0 * (off + pos0), axis=1, keepdims=True)
    dest1 = jnp.sum(oh1 * (off + c0 + pos1), axis=1, keepdims=True)

    q = pad - cnt
    qx = lax.dot_general(q, upper, dots, preferred_element_type=jnp.float32)
    q8 = jnp.sum(q, axis=1, keepdims=True)
    j = lax.broadcasted_iota(jnp.int32, (T, 1), 0).astype(jnp.float32)
    in_e = ((j >= qx) & (j < qx + q)).astype(jnp.float32)
    dp_in = jnp.sum(in_e * (off + cnt + j - qx), axis=1, keepdims=True)
    dest_pad = jnp.where(j < q8, dp_in, float(2 * T) + j)

    dest_ref[...] = jnp.concatenate(
        [dest0, dest1, dest_pad], axis=1).astype(jnp.int32)
    w_ref[...] = jnp.concatenate(
        [nw0, nw1, jnp.zeros_like(nw0)], axis=1)

    sb = lax.broadcasted_iota(jnp.int32, (NBLK, 1), 0).astype(jnp.float32) * BLK
    blk = jnp.sum((sb >= off).astype(jnp.int32), axis=1, keepdims=True) - 1
    blk_ref[...] = jnp.clip(blk, 0, E - 1)


def _gating(x_flat, gate_w, gate_b):
    return pl.pallas_call(
        _gating_body,
        out_shape=(
            jax.ShapeDtypeStruct((T, 3), jnp.int32),
            jax.ShapeDtypeStruct((T, 3), jnp.float32),
            jax.ShapeDtypeStruct((NBLK, 1), jnp.int32),
        ),
    )(x_flat, gate_w, gate_b.reshape(1, E))


def _dispatch(xb3, dest3, destf, tok3, w_rep):
    mesh = plsc.VectorSubcoreMesh(core_axis_name="c", subcore_axis_name="s")

    @functools.partial(
        pl.kernel, mesh=mesh,
        out_type=(jax.ShapeDtypeStruct((NPAD, 8, 128), jnp.float32),
                  jax.ShapeDtypeStruct((NPAD, WLN), jnp.float32)),
        scratch_types=[pltpu.VMEM((NCH, WCH), jnp.int32),
                       pltpu.VMEM((NCH, WCH), jnp.int32),
                       pltpu.VMEM((PER,), jnp.int32),
                       pltpu.VMEM((WCH, 8, 128), jnp.float32),
                       pltpu.VMEM((WCH, 8, 128), jnp.float32),
                       pltpu.VMEM((PER, WLN), jnp.float32),
                       pltpu.SemaphoreType.DMA,
                       pltpu.SemaphoreType.DMA,
                       pltpu.SemaphoreType.DMA,
                       pltpu.SemaphoreType.DMA,
                       pltpu.SemaphoreType.DMA])
    def k(x_hbm, dest_hbm, destf_hbm, tok_hbm, w_hbm, xs_hbm, sw_hbm,
          tid_v, did_v, dif_v, buf_a, buf_b, wbuf, ga, gb, sa, sb_, ws):
        wid = lax.axis_index("s") * NC + lax.axis_index("c")
        pltpu.sync_copy(tok_hbm.at[wid], tid_v)
        pltpu.sync_copy(dest_hbm.at[wid], did_v)
        pltpu.sync_copy(destf_hbm.at[wid], dif_v)
        pltpu.sync_copy(w_hbm.at[wid], wbuf)
        wsc = pltpu.async_copy(wbuf, sw_hbm.at[dif_v], ws)

        bufs = (buf_a, buf_b)
        gsem = (ga, gb)
        ssem = (sa, sb_)
        gh = [None] * NCH
        sh = [None] * NCH
        gh[0] = pltpu.async_copy(x_hbm.at[tid_v.at[0]], bufs[0], gsem[0])
        for kk in range(NCH):
            cur = kk % 2
            gh[kk].wait()
            if kk + 1 < NCH:
                nxt = (kk + 1) % 2
                if kk >= 1:
                    sh[kk - 1].wait()
                gh[kk + 1] = pltpu.async_copy(
                    x_hbm.at[tid_v.at[kk + 1]], bufs[nxt], gsem[nxt])
            sh[kk] = pltpu.async_copy(bufs[cur], xs_hbm.at[did_v.at[kk]],
                                      ssem[cur])
        sh[NCH - 2].wait()
        sh[NCH - 1].wait()
        wsc.wait()

    return k(xb3, dest3, destf, tok3, w_rep)


_GA1, _GA2, _GA3 = 0.254829592, -0.284496736, 1.421413741
_GA4, _GA5, _GP = -1.453152027, 1.061405429, 0.3275911


def _erf(z):
    s = jnp.sign(z)
    az = jnp.abs(z)
    t = 1.0 / (1.0 + _GP * az)
    poly = ((((_GA5 * t + _GA4) * t + _GA3) * t + _GA2) * t + _GA1) * t
    return s * (1.0 - poly * jnp.exp(-az * az))


def _ffn_body(blk_ref, xs_ref, sw_ref, w1_ref, b1_ref, w2_ref, b2_ref,
              ys_ref):
    h_idx = pl.program_id(1)
    hb = (lax.dot_general(
        xs_ref[...].astype(jnp.bfloat16), w1_ref[0], (((1,), (0,)), ((), ())),
        preferred_element_type=jnp.float32) + b1_ref[0, 0]
          ).astype(jnp.bfloat16)
    half = jnp.bfloat16(0.5)
    one = jnp.bfloat16(1.0)
    act = half * hb * (one + _erf(hb * jnp.bfloat16(0.7071067811865476)))
    part = lax.dot_general(
        act, w2_ref[0], (((1,), (0,)), ((), ())),
        preferred_element_type=jnp.float32)

    @pl.when(h_idx == 0)
    def _():
        ys_ref[...] = part + b2_ref[0]

    @pl.when(h_idx > 0)
    def _():
        ys_ref[...] += part

    @pl.when(h_idx == NH - 1)
    def _():
        ys_ref[...] *= sw_ref[...][:, :1]


def _ffn(blk_e, xs, sw, w1b, b1, w2b, b2):
    grid_spec = pltpu.PrefetchScalarGridSpec(
        num_scalar_prefetch=1,
        grid=(NBLK, NH),
        in_specs=[
            pl.BlockSpec((BLK, C), lambda b, h, be: (b, 0)),
            pl.BlockSpec((BLK, WLN), lambda b, h, be: (b, 0)),
            pl.BlockSpec((1, C, HT), lambda b, h, be: (be[b], 0, h)),
            pl.BlockSpec((1, 1, 1, HT), lambda b, h, be: (be[b], h, 0, 0)),
            pl.BlockSpec((1, HT, C), lambda b, h, be: (be[b], h, 0)),
            pl.BlockSpec((1, 1, C), lambda b, h, be: (be[b], 0, 0)),
        ],
        out_specs=pl.BlockSpec((BLK, C), lambda b, h, be: (b, 0)),
    )
    return pl.pallas_call(
        _ffn_body,
        grid_spec=grid_spec,
        out_shape=jax.ShapeDtypeStruct((NPAD, C), jnp.float32),
    )(blk_e, xs, sw, w1b, b1.reshape(E, NH, 1, HT), w2b, b2.reshape(E, 1, C))


def _combine(ys, d0, d1):
    mesh = plsc.VectorSubcoreMesh(core_axis_name="c", subcore_axis_name="s")

    @functools.partial(
        pl.kernel, mesh=mesh,
        out_type=jax.ShapeDtypeStruct((T, C), jnp.float32),
        scratch_types=[pltpu.VMEM((WCH,), jnp.int32),
                       pltpu.VMEM((WCH,), jnp.int32),
                       pltpu.VMEM((WCH, C), jnp.float32),
                       pltpu.VMEM((WCH, C), jnp.float32),
                       pltpu.SemaphoreType.DMA])
    def k(ys_hbm, d0_hbm, d1_hbm, out_hbm, i0_v, i1_v, g0, g1, sem):
        wid = lax.axis_index("s") * NC + lax.axis_index("c")
        per = T // NW
        for kk in range(per // WCH):
            base = wid * per + kk * WCH
            pltpu.sync_copy(d0_hbm.at[0, pl.ds(base, WCH)], i0_v)
            pltpu.sync_copy(d1_hbm.at[0, pl.ds(base, WCH)], i1_v)
            pltpu.async_copy(ys_hbm.at[i0_v], g0, sem).wait()
            pltpu.async_copy(ys_hbm.at[i1_v], g1, sem).wait()

            @pl.loop(0, WCH)
            def _(r):
                for cc in range(0, C, 16):
                    g0[r, pl.ds(cc, 16)] = (g0[r, pl.ds(cc, 16)]
                                            + g1[r, pl.ds(cc, 16)])

            pltpu.sync_copy(g0, out_hbm.at[pl.ds(base, WCH)])

    return k(ys, d0, d1)


def kernel(x, gate_w, gate_b, w1, b1, w2, b2):
    B, Tb, Cb = x.shape
    x_flat = x.reshape(T, C)

    dest_out, w_out, blk_e = _gating(x_flat, gate_w, gate_b)

    dest_flat = jnp.concatenate(
        [dest_out[:, 0], dest_out[:, 1], dest_out[:NPF, 2]])
    tok = jnp.arange(T, dtype=jnp.int32)
    tok_flat = jnp.concatenate([tok, tok, jnp.zeros((NPF,), jnp.int32)])
    w_flat = jnp.concatenate(
        [w_out[:, 0], w_out[:, 1], jnp.zeros((NPF,), jnp.float32)])

    xb3 = x_flat.reshape(T, 8, 128)
    xs, sw = _dispatch(xb3,
                       dest_flat.reshape(NW, NCH, WCH),
                       dest_flat.reshape(NW, PER),
                       tok_flat.reshape(NW, NCH, WCH),
                       jnp.broadcast_to(w_flat.reshape(NPAD, 1),
                                        (NPAD, WLN)).reshape(NW, PER, WLN))

    ys = _ffn(blk_e.reshape(NBLK), xs.reshape(NPAD, C), sw,
              w1.astype(jnp.bfloat16), b1, w2.astype(jnp.bfloat16), b2)

    d0 = dest_out[:, 0].reshape(1, T)
    d1 = dest_out[:, 1].reshape(1, T)
    out = _combine(ys, d0, d1)
    return out.reshape(B, Tb, Cb)


# speedup vs baseline: 1.3335x; 1.0481x over previous
import functools

import jax
import jax.numpy as jnp
from jax import lax
from jax.experimental import pallas as pl
from jax.experimental.pallas import tpu as pltpu
from jax.experimental.pallas import tpu_sc as plsc

T = 2048
C = 1024
E = 8
H = 4096
BLK = 256
NBLK = 24
NPAD = NBLK * BLK
NPF = NPAD - 2 * T
NH = 4
HT = H // NH
NC, NS = 2, 16
NW = NC * NS
PER = NPAD // NW
WCH = 32
NCH = PER // WCH
WLN = 128


def _gating_body(x_ref, gw_ref, gb_ref, dest_ref, wrep_ref, blk_ref):
    logits = lax.dot_general(
        x_ref[...].astype(jnp.bfloat16), gw_ref[...].astype(jnp.bfloat16),
        (((1,), (0,)), ((), ())),
        preferred_element_type=jnp.float32) + gb_ref[...]

    it8 = lax.broadcasted_iota(jnp.int32, (T, E), 1)
    m1 = jnp.max(logits, axis=1, keepdims=True)
    i1 = jnp.min(jnp.where(logits == m1, it8, E), axis=1, keepdims=True)
    l2 = jnp.where(it8 == i1, -jnp.inf, logits)
    m2 = jnp.max(l2, axis=1, keepdims=True)
    i2 = jnp.min(jnp.where(l2 == m2, it8, E), axis=1, keepdims=True)

    t = jnp.exp(m2 - m1)
    nw0 = 1.0 / (1.0 + t)
    nw1 = t / (1.0 + t)

    oh0 = (it8 == i1).astype(jnp.float32)
    oh1 = (it8 == i2).astype(jnp.float32)

    r_io = lax.broadcasted_iota(jnp.int32, (T, T), 0)
    c_io = lax.broadcasted_iota(jnp.int32, (T, T), 1)
    tri = (r_io > c_io).astype(jnp.float32)
    dots = (((1,), (0,)), ((), ()))
    pos0 = lax.dot_general(tri, oh0, dots, preferred_element_type=jnp.float32)
    pos1 = lax.dot_general(tri, oh1, dots, preferred_element_type=jnp.float32)

    c0 = jnp.sum(oh0, axis=0, keepdims=True)
    c1 = jnp.sum(oh1, axis=0, keepdims=True)
    cnt = c0 + c1
    pad = jnp.ceil(cnt / BLK) * BLK

    e_r = lax.broadcasted_iota(jnp.int32, (E, E), 0)
    e_c = lax.broadcasted_iota(jnp.int32, (E, E), 1)
    upper = (e_r < e_c).astype(jnp.float32)
    off = lax.dot_general(pad, upper, dots, preferred_element_type=jnp.float32)

    dest0 = jnp.sum(oh0 * (off + pos0), axis=1, keepdims=True)
    dest1 = jnp.sum(oh1 * (off + c0 + pos1), axis=1, keepdims=True)

    q = pad - cnt
    qx = lax.dot_general(q, upper, dots, preferred_element_type=jnp.float32)
    q8 = jnp.sum(q, axis=1, keepdims=True)
    j = lax.broadcasted_iota(jnp.int32, (T, 1), 0).astype(jnp.float32)
    in_e = ((j >= qx) & (j < qx + q)).astype(jnp.float32)
    dp_in = jnp.sum(in_e * (off + cnt + j - qx), axis=1, keepdims=True)
    dest_pad = jnp.where(j < q8, dp_in, float(2 * T) + j)

    dest_ref[...] = jnp.concatenate(
        [dest0, dest1, dest_pad], axis=1).astype(jnp.int32)
    wrep_ref[0:T, :] = jnp.broadcast_to(nw0, (T, WLN))
    wrep_ref[T:2 * T, :] = jnp.broadcast_to(nw1, (T, WLN))
    wrep_ref[2 * T:, :] = jnp.zeros((NPF, WLN), jnp.float32)

    sb = lax.broadcasted_iota(jnp.int32, (NBLK, 1), 0).astype(jnp.float32) * BLK
    blk = jnp.sum((sb >= off).astype(jnp.int32), axis=1, keepdims=True) - 1
    blk_ref[...] = jnp.clip(blk, 0, E - 1)


def _gating(x_flat, gate_w, gate_b):
    return pl.pallas_call(
        _gating_body,
        out_shape=(
            jax.ShapeDtypeStruct((T, 3), jnp.int32),
            jax.ShapeDtypeStruct((NPAD, WLN), jnp.float32),
            jax.ShapeDtypeStruct((NBLK, 1), jnp.int32),
        ),
    )(x_flat, gate_w, gate_b.reshape(1, E))


def _dispatch(xb3, dest3, destf, tok3, w_rep):
    mesh = plsc.VectorSubcoreMesh(core_axis_name="c", subcore_axis_name="s")

    @functools.partial(
        pl.kernel, mesh=mesh,
        out_type=(jax.ShapeDtypeStruct((NPAD, C), jnp.float32),
                  jax.ShapeDtypeStruct((NPAD, WLN), jnp.float32)),
        scratch_types=[pltpu.VMEM((NCH, WCH), jnp.int32),
                       pltpu.VMEM((NCH, WCH), jnp.int32),
                       pltpu.VMEM((PER,), jnp.int32),
                       pltpu.VMEM((WCH, C), jnp.float32),
                       pltpu.VMEM((WCH, C), jnp.float32),
                       pltpu.VMEM((PER, WLN), jnp.float32),
                       pltpu.SemaphoreType.DMA,
                       pltpu.SemaphoreType.DMA,
                       pltpu.SemaphoreType.DMA,
                       pltpu.SemaphoreType.DMA,
                       pltpu.SemaphoreType.DMA])
    def k(x_hbm, dest_hbm, destf_hbm, tok_hbm, w_hbm, xs_hbm, sw_hbm,
          tid_v, did_v, dif_v, buf_a, buf_b, wbuf, ga, gb, sa, sb_, ws):
        wid = lax.axis_index("s") * NC + lax.axis_index("c")
        pltpu.sync_copy(tok_hbm.at[wid], tid_v)
        pltpu.sync_copy(dest_hbm.at[wid], did_v)
        pltpu.sync_copy(destf_hbm.at[wid], dif_v)
        pltpu.sync_copy(w_hbm.at[wid], wbuf)
        wsc = pltpu.async_copy(wbuf, sw_hbm.at[dif_v], ws)

        bufs = (buf_a, buf_b)
        gsem = (ga, gb)
        ssem = (sa, sb_)
        gh = [None] * NCH
        sh = [None] * NCH
        gh[0] = pltpu.async_copy(x_hbm.at[tid_v.at[0]], bufs[0], gsem[0])
        for kk in range(NCH):
            cur = kk % 2
            gh[kk].wait()
            if kk + 1 < NCH:
                nxt = (kk + 1) % 2
                if kk >= 1:
                    sh[kk - 1].wait()
                gh[kk + 1] = pltpu.async_copy(
                    x_hbm.at[tid_v.at[kk + 1]], bufs[nxt], gsem[nxt])
            sh[kk] = pltpu.async_copy(bufs[cur], xs_hbm.at[did_v.at[kk]],
                                      ssem[cur])
        sh[NCH - 2].wait()
        sh[NCH - 1].wait()
        wsc.wait()

    return k(xb3, dest3, destf, tok3, w_rep)


_GA1, _GA2, _GA3 = 0.254829592, -0.284496736, 1.421413741
_GA4, _GA5, _GP = -1.453152027, 1.061405429, 0.3275911


def _erf(z):
    s = jnp.sign(z)
    az = jnp.abs(z)
    t = 1.0 / (1.0 + _GP * az)
    poly = ((((_GA5 * t + _GA4) * t + _GA3) * t + _GA2) * t + _GA1) * t
    return s * (1.0 - poly * jnp.exp(-az * az))


def _ffn_body(blk_ref, xs_ref, sw_ref, w1_ref, b1_ref, w2_ref, b2_ref,
              ys_ref):
    h_idx = pl.program_id(1)
    hb = (lax.dot_general(
        xs_ref[...].astype(jnp.bfloat16), w1_ref[0], (((1,), (0,)), ((), ())),
        preferred_element_type=jnp.float32) + b1_ref[0, 0]
          ).astype(jnp.bfloat16)
    half = jnp.bfloat16(0.5)
    one = jnp.bfloat16(1.0)
    act = half * hb * (one + _erf(hb * jnp.bfloat16(0.7071067811865476)))
    part = lax.dot_general(
        act, w2_ref[0], (((1,), (0,)), ((), ())),
        preferred_element_type=jnp.float32)

    @pl.when(h_idx == 0)
    def _():
        ys_ref[...] = part + b2_ref[0]

    @pl.when(h_idx > 0)
    def _():
        ys_ref[...] += part

    @pl.when(h_idx == NH - 1)
    def _():
        ys_ref[...] *= sw_ref[...][:, :1]


def _ffn(blk_e, xs, sw, w1b, b1, w2b, b2):
    grid_spec = pltpu.PrefetchScalarGridSpec(
        num_scalar_prefetch=1,
        grid=(NBLK, NH),
        in_specs=[
            pl.BlockSpec((BLK, C), lambda b, h, be: (b, 0)),
            pl.BlockSpec((BLK, WLN), lambda b, h, be: (b, 0)),
            pl.BlockSpec((1, C, HT), lambda b, h, be: (be[b], 0, h)),
            pl.BlockSpec((1, 1, 1, HT), lambda b, h, be: (be[b], h, 0, 0)),
            pl.BlockSpec((1, HT, C), lambda b, h, be: (be[b], h, 0)),
            pl.BlockSpec((1, 1, C), lambda b, h, be: (be[b], 0, 0)),
        ],
        out_specs=pl.BlockSpec((BLK, C), lambda b, h, be: (b, 0)),
    )
    return pl.pallas_call(
        _ffn_body,
        grid_spec=grid_spec,
        out_shape=jax.ShapeDtypeStruct((NPAD, C), jnp.float32),
    )(blk_e, xs, sw, w1b, b1.reshape(E, NH, 1, HT), w2b, b2.reshape(E, 1, C))


def _combine(ys, d0, d1):
    mesh = plsc.VectorSubcoreMesh(core_axis_name="c", subcore_axis_name="s")

    @functools.partial(
        pl.kernel, mesh=mesh,
        out_type=jax.ShapeDtypeStruct((T, C), jnp.float32),
        scratch_types=[pltpu.VMEM((WCH,), jnp.int32),
                       pltpu.VMEM((WCH,), jnp.int32),
                       pltpu.VMEM((WCH, C), jnp.float32),
                       pltpu.VMEM((WCH, C), jnp.float32),
                       pltpu.SemaphoreType.DMA])
    def k(ys_hbm, d0_hbm, d1_hbm, out_hbm, i0_v, i1_v, g0, g1, sem):
        wid = lax.axis_index("s") * NC + lax.axis_index("c")
        per = T // NW
        for kk in range(per // WCH):
            base = wid * per + kk * WCH
            pltpu.sync_copy(d0_hbm.at[0, pl.ds(base, WCH)], i0_v)
            pltpu.sync_copy(d1_hbm.at[0, pl.ds(base, WCH)], i1_v)
            pltpu.async_copy(ys_hbm.at[i0_v], g0, sem).wait()
            pltpu.async_copy(ys_hbm.at[i1_v], g1, sem).wait()

            @pl.loop(0, WCH)
            def _(r):
                for cc in range(0, C, 16):
                    g0[r, pl.ds(cc, 16)] = (g0[r, pl.ds(cc, 16)]
                                            + g1[r, pl.ds(cc, 16)])

            pltpu.sync_copy(g0, out_hbm.at[pl.ds(base, WCH)])

    return k(ys, d0, d1)


def kernel(x, gate_w, gate_b, w1, b1, w2, b2):
    B, Tb, Cb = x.shape
    x_flat = x.reshape(T, C)

    dest_out, w_rep, blk_e = _gating(x_flat, gate_w, gate_b)

    dest_flat = jnp.concatenate(
        [dest_out[:, 0], dest_out[:, 1], dest_out[:NPF, 2]])
    tok = jnp.arange(T, dtype=jnp.int32)
    tok_flat = jnp.concatenate([tok, tok, jnp.zeros((NPF,), jnp.int32)])

    xs, sw = _dispatch(x_flat,
                       dest_flat.reshape(NW, NCH, WCH),
                       dest_flat.reshape(NW, PER),
                       tok_flat.reshape(NW, NCH, WCH),
                       w_rep.reshape(NW, PER, WLN))

    ys = _ffn(blk_e.reshape(NBLK), xs, sw,
              w1.astype(jnp.bfloat16), b1, w2.astype(jnp.bfloat16), b2)

    d0 = dest_out[:, 0].reshape(1, T)
    d1 = dest_out[:, 1].reshape(1, T)
    out = _combine(ys, d0, d1)
    return out.reshape(B, Tb, Cb)
